# baseline (speedup 1.0000x reference)
import functools

import jax
import jax.numpy as jnp
from jax.experimental import pallas as pl
from jax.experimental.pallas import tpu as pltpu

EPS = 1e-5


def _shifted(xf, s, cin, hw):
    if s == 0:
        return xf
    if s > 0:
        return jnp.concatenate(
            [xf[:, s:], jnp.zeros((cin, s), xf.dtype)], axis=1)
    return jnp.concatenate(
        [jnp.zeros((cin, -s), xf.dtype), xf[:, :hw + s]], axis=1)


def _conv_stats_kernel(x_ref, w_ref, y_ref, stats_ref, *, h_out, w_out,
                       kh_size, kw_size):
    cin = x_ref.shape[1]
    hw = h_out * w_out
    xf = x_ref[0].astype(jnp.bfloat16).reshape(cin, hw)
    col = jax.lax.broadcasted_iota(jnp.int32, (1, hw), 1) % w_out
    acc = None
    for kh in range(kh_size):
        dr = kh - (kh_size // 2)
        for kw in range(kw_size):
            dc = kw - (kw_size // 2)
            xs = _shifted(xf, dr * w_out + dc, cin, hw)
            if dc < 0:
                xs = jnp.where(col >= -dc, xs, jnp.bfloat16(0))
            elif dc > 0:
                xs = jnp.where(col < w_out - dc, xs, jnp.bfloat16(0))
            part = jax.lax.dot_general(
                w_ref[kh * kw_size + kw], xs,
                dimension_numbers=(((1,), (0,)), ((), ())),
                preferred_element_type=jnp.float32)
            acc = part if acc is None else acc + part
    psum = jnp.sum(acc, axis=1, keepdims=True)
    psq = jnp.sum(acc * acc, axis=1, keepdims=True)
    stats_ref[0] = jnp.concatenate([psum, psq], axis=1)
    y_ref[0] = acc.astype(jnp.bfloat16)


def _bn_relu_kernel(y_ref, scale_ref, shift_ref, o_ref, *, h_out, w_out):
    z = jnp.maximum(
        y_ref[0].astype(jnp.float32) * scale_ref[...] + shift_ref[...], 0.0)
    o_ref[0] = z.reshape(z.shape[0], h_out, w_out)


def kernel(x_nchw, w_hwio, bias, gamma, beta):
    del bias
    N, Cin, H, W = x_nchw.shape
    KH, KW, _, Cout = w_hwio.shape
    HW = H * W

    w_t = jnp.transpose(w_hwio.reshape(KH * KW, Cin, Cout),
                        (0, 2, 1)).astype(jnp.bfloat16)

    cparams = pltpu.CompilerParams(
        dimension_semantics=("parallel",),
        vmem_limit_bytes=64 * 1024 * 1024)

    conv_flops = 2 * N * HW * KH * KW * Cin * Cout
    y, stats = pl.pallas_call(
        functools.partial(_conv_stats_kernel, h_out=H, w_out=W, kh_size=KH,
                          kw_size=KW),
        grid=(N,),
        in_specs=[
            pl.BlockSpec((1, Cin, H, W), lambda n: (n, 0, 0, 0)),
            pl.BlockSpec((KH * KW, Cout, Cin), lambda n: (0, 0, 0)),
        ],
        out_specs=[
            pl.BlockSpec((1, Cout, HW), lambda n: (n, 0, 0)),
            pl.BlockSpec((1, Cout, 2), lambda n: (n, 0, 0)),
        ],
        out_shape=[
            jax.ShapeDtypeStruct((N, Cout, HW), jnp.bfloat16),
            jax.ShapeDtypeStruct((N, Cout, 2), jnp.float32),
        ],
        compiler_params=cparams,
        cost_estimate=pl.CostEstimate(
            flops=int(conv_flops + 4 * N * HW * Cout),
            transcendentals=0,
            bytes_accessed=int(4 * x_nchw.size + 2 * w_t.size
                               + 2 * N * HW * Cout + 4 * N * 2 * Cout)),
    )(x_nchw, w_t)

    count = float(N * HW)
    total = jnp.sum(stats, axis=0)
    mean = total[:, 0] / count
    var = total[:, 1] / count - mean * mean
    inv_std = jax.lax.rsqrt(var + EPS)
    scale = (gamma.astype(jnp.float32) * inv_std).reshape(Cout, 1)
    shift = (beta.astype(jnp.float32) - mean * scale[:, 0]).reshape(Cout, 1)

    out = pl.pallas_call(
        functools.partial(_bn_relu_kernel, h_out=H, w_out=W),
        grid=(N,),
        in_specs=[
            pl.BlockSpec((1, Cout, HW), lambda n: (n, 0, 0)),
            pl.BlockSpec((Cout, 1), lambda n: (0, 0)),
            pl.BlockSpec((Cout, 1), lambda n: (0, 0)),
        ],
        out_specs=pl.BlockSpec((1, Cout, H, W), lambda n: (n, 0, 0, 0)),
        out_shape=jax.ShapeDtypeStruct((N, Cout, H, W), jnp.float32),
        compiler_params=cparams,
        cost_estimate=pl.CostEstimate(
            flops=int(3 * N * HW * Cout),
            transcendentals=0,
            bytes_accessed=int(2 * N * HW * Cout + 4 * N * HW * Cout
                               + 8 * Cout)),
    )(y, scale, shift)

    return out


# speedup vs baseline: 1.5633x; 1.5633x over previous
import functools

import jax
import jax.numpy as jnp
from jax.experimental import pallas as pl
from jax.experimental.pallas import tpu as pltpu

EPS = 1e-5
LANES = 128


def _round_up(x, m):
    return (x + m - 1) // m * m


def _conv_stats_kernel(x_ref, w_ref, y_ref, stats_ref, *, h_out, w_out,
                       kh_size, kw_size):
    h, w = h_out, w_out
    c = x_ref.shape[3]
    rows = h * w
    img = x_ref[0].astype(jnp.bfloat16)
    zw = jnp.zeros((h, 1, c), jnp.bfloat16)
    imgw = jnp.concatenate([zw, img, zw], axis=1)
    zh = jnp.zeros((1, w + 2, c), jnp.bfloat16)
    slab = jnp.concatenate([zh, imgw, zh], axis=0)
    acc = None
    for kh in range(kh_size):
        row_slab = slab[kh:kh + h]
        for kw in range(kw_size):
            win = row_slab[:, kw:kw + w, :].reshape(rows, c)
            part = jax.lax.dot_general(
                win, w_ref[kh * kw_size + kw],
                dimension_numbers=(((1,), (0,)), ((), ())),
                preferred_element_type=jnp.float32)
            acc = part if acc is None else acc + part
    ones_r = jnp.ones((8, rows), jnp.float32)
    dn = (((1,), (0,)), ((), ()))
    psum = jax.lax.dot_general(ones_r, acc, dn,
                               preferred_element_type=jnp.float32)
    psq = jax.lax.dot_general(ones_r, acc * acc, dn,
                              preferred_element_type=jnp.float32)
    stats_ref[0] = jnp.concatenate([psum[0:1], psq[0:1]], axis=0)
    y_ref[0] = acc[:, :y_ref.shape[2]].astype(jnp.bfloat16)


def _bn_relu_kernel(y_ref, scale_ref, shift_ref, o_ref, *, h_out, w_out):
    z = jnp.maximum(
        y_ref[0].astype(jnp.float32) * scale_ref[...] + shift_ref[...], 0.0)
    o_ref[0] = z.reshape(h_out, w_out, z.shape[1])


def kernel(x_nchw, w_hwio, bias, gamma, beta):
    del bias
    N, Cin, H, W = x_nchw.shape
    KH, KW, _, Cout = w_hwio.shape
    CPAD = _round_up(Cout, LANES)
    HW = H * W

    x_nhwc = jnp.transpose(x_nchw, (0, 2, 3, 1))
    w_packed = jnp.pad(
        w_hwio.reshape(KH * KW, Cin, Cout),
        ((0, 0), (0, 0), (0, CPAD - Cout))).astype(jnp.bfloat16)

    cparams = pltpu.CompilerParams(
        dimension_semantics=("parallel",),
        vmem_limit_bytes=64 * 1024 * 1024)

    conv_flops = 2 * N * HW * KH * KW * Cin * CPAD
    y, stats = pl.pallas_call(
        functools.partial(_conv_stats_kernel, h_out=H, w_out=W, kh_size=KH,
                          kw_size=KW),
        grid=(N,),
        in_specs=[
            pl.BlockSpec((1, H, W, Cin), lambda n: (n, 0, 0, 0)),
            pl.BlockSpec((KH * KW, Cin, CPAD), lambda n: (0, 0, 0)),
        ],
        out_specs=[
            pl.BlockSpec((1, HW, Cout), lambda n: (n, 0, 0)),
            pl.BlockSpec((1, 2, CPAD), lambda n: (n, 0, 0)),
        ],
        out_shape=[
            jax.ShapeDtypeStruct((N, HW, Cout), jnp.bfloat16),
            jax.ShapeDtypeStruct((N, 2, CPAD), jnp.float32),
        ],
        compiler_params=cparams,
        cost_estimate=pl.CostEstimate(
            flops=int(conv_flops + 4 * N * HW * CPAD),
            transcendentals=0,
            bytes_accessed=int(4 * x_nhwc.size + 2 * w_packed.size
                               + 2 * N * HW * Cout + 4 * N * 2 * CPAD)),
    )(x_nhwc, w_packed)

    count = float(N * HW)
    total = jnp.sum(stats, axis=0)
    mean = total[0, :Cout] / count
    var = total[1, :Cout] / count - mean * mean
    inv_std = jax.lax.rsqrt(var + EPS)
    scale = (gamma.astype(jnp.float32) * inv_std).reshape(1, Cout)
    shift = (beta.astype(jnp.float32) - mean * scale[0]).reshape(1, Cout)

    out = pl.pallas_call(
        functools.partial(_bn_relu_kernel, h_out=H, w_out=W),
        grid=(N,),
        in_specs=[
            pl.BlockSpec((1, HW, Cout), lambda n: (n, 0, 0)),
            pl.BlockSpec((1, Cout), lambda n: (0, 0)),
            pl.BlockSpec((1, Cout), lambda n: (0, 0)),
        ],
        out_specs=pl.BlockSpec((1, H, W, Cout), lambda n: (n, 0, 0, 0)),
        out_shape=jax.ShapeDtypeStruct((N, H, W, Cout), jnp.float32),
        compiler_params=cparams,
        cost_estimate=pl.CostEstimate(
            flops=int(3 * N * HW * Cout),
            transcendentals=0,
            bytes_accessed=int(2 * N * HW * Cout + 4 * N * HW * Cout
                               + 8 * Cout)),
    )(y, scale, shift)

    return jnp.transpose(out, (0, 3, 1, 2))
